# baseline (speedup 1.0000x reference)
import functools

import jax
import jax.numpy as jnp
from jax import lax
from jax.experimental import pallas as pl
from jax.experimental.pallas import tpu as pltpu
from jax.experimental.pallas import tpu_sc as plsc

N = 10000
F = 128
OUT = 64
N_PAD = 10240
NC, NS = 2, 16
NW = NC * NS
ROWS_PER_TILE = N_PAD // NS
CHUNK = 128
DEG_W = F

_MESH = plsc.VectorSubcoreMesh(core_axis_name="c", subcore_axis_name="s")


def _make_deg(n_chunks):
    cpp = n_chunks // PHASES

    @functools.partial(
        pl.kernel,
        out_type=jax.ShapeDtypeStruct((NC, N_PAD, DEG_W), jnp.float32),
        mesh=_MESH,
        scratch_types=[
            pltpu.VMEM_SHARED((N_PAD, DEG_W), jnp.float32),
            pltpu.VMEM((cpp, CHUNK), jnp.int32),
            pltpu.VMEM((CHUNK, DEG_W), jnp.float32),
        ],
    )
    def deg_kernel(dst_hbm, zeros_hbm, ones_hbm, out_hbm, acc, dst_v, ones_v):
        cid = lax.axis_index("c")
        sid = lax.axis_index("s")
        wid = sid * NC + cid
        row0 = sid * ROWS_PER_TILE
        pltpu.sync_copy(zeros_hbm.at[pl.ds(row0, ROWS_PER_TILE)],
                        acc.at[pl.ds(row0, ROWS_PER_TILE)])
        pltpu.sync_copy(ones_hbm, ones_v)
        plsc.subcore_barrier()

        def body(j, carry):
            pltpu.sync_copy(ones_v, acc.at[dst_v.at[j]], add=True)
            return carry

        for p in range(PHASES):
            pltpu.sync_copy(dst_hbm.at[wid, p], dst_v)
            lax.fori_loop(0, cpp, body, 0)
        plsc.subcore_barrier()
        pltpu.sync_copy(acc.at[pl.ds(row0, ROWS_PER_TILE)],
                        out_hbm.at[cid, pl.ds(row0, ROWS_PER_TILE)])

    return deg_kernel


KG = 2
PHASES = 2


def _make_prop(n_chunks):
    assert n_chunks % (PHASES * KG) == 0
    cpp = n_chunks // PHASES
    n_pairs = cpp // KG

    @functools.partial(
        pl.kernel,
        out_type=jax.ShapeDtypeStruct((NC, N_PAD, F), jnp.float32),
        mesh=_MESH,
        scratch_types=[
            pltpu.VMEM_SHARED((N_PAD, F), jnp.float32),
            pltpu.VMEM((cpp * CHUNK,), jnp.int32),
            pltpu.VMEM((cpp, CHUNK), jnp.int32),
            pltpu.VMEM((KG * CHUNK, F), jnp.float32),
            pltpu.SemaphoreType.DMA,
        ],
    )
    def prop_kernel(t_hbm, srcf_hbm, dst_hbm, zeros_hbm, out_hbm,
                    acc, src_v, dst_v, rows_v, sem):
        cid = lax.axis_index("c")
        sid = lax.axis_index("s")
        wid = sid * NC + cid
        row0 = sid * ROWS_PER_TILE
        pltpu.sync_copy(zeros_hbm.at[pl.ds(row0, ROWS_PER_TILE)],
                        acc.at[pl.ds(row0, ROWS_PER_TILE)])
        plsc.subcore_barrier()

        def body(j, carry):
            pltpu.async_copy(t_hbm.at[src_v.at[pl.ds(j * (KG * CHUNK),
                                                     KG * CHUNK)]],
                             rows_v, sem).wait()
            for b in range(KG):
                pltpu.sync_copy(rows_v.at[pl.ds(b * CHUNK, CHUNK)],
                                acc.at[dst_v.at[j * KG + b]], add=True)
            return carry

        for p in range(PHASES):
            pltpu.sync_copy(srcf_hbm.at[wid, p], src_v)
            pltpu.sync_copy(dst_hbm.at[wid, p], dst_v)
            lax.fori_loop(0, n_pairs, body, 0)

        plsc.subcore_barrier()
        pltpu.sync_copy(acc.at[pl.ds(row0, ROWS_PER_TILE)],
                        out_hbm.at[cid, pl.ds(row0, ROWS_PER_TILE)])

    return prop_kernel


RB = 1280
GRID = N_PAD // RB


def _tc0_body(x_ref, w_ref, p_ref, t_ref, dinv_ref):
    p = p_ref[...]
    deg = 1.0 + p[0, :, 0:1] + p[1, :, 0:1]
    dinvb = jnp.broadcast_to(lax.rsqrt(deg), (RB, F))
    t_ref[...] = dinvb * jnp.dot(x_ref[...], w_ref[...],
                                 preferred_element_type=jnp.float32)
    dinv_ref[...] = dinvb


_tc0 = pl.pallas_call(
    _tc0_body,
    grid=(GRID,),
    in_specs=[
        pl.BlockSpec((RB, F), lambda i: (i, 0)),
        pl.BlockSpec((F, F), lambda i: (0, 0)),
        pl.BlockSpec((NC, RB, DEG_W), lambda i: (0, i, 0)),
    ],
    out_specs=[
        pl.BlockSpec((RB, F), lambda i: (i, 0)),
        pl.BlockSpec((RB, F), lambda i: (i, 0)),
    ],
    out_shape=[
        jax.ShapeDtypeStruct((N_PAD, F), jnp.float32),
        jax.ShapeDtypeStruct((N_PAD, F), jnp.float32),
    ],
)


def _tc_mid_body(t_ref, p_ref, dinv_ref, b_ref, w_ref, out_ref):
    s = t_ref[...] + p_ref[0] + p_ref[1]
    pre = dinv_ref[...] * s + b_ref[...]
    h = jnp.tanh(pre)
    out_ref[...] = dinv_ref[...] * jnp.dot(h, w_ref[...],
                                           preferred_element_type=jnp.float32)


_tc_mid = pl.pallas_call(
    _tc_mid_body,
    grid=(GRID,),
    in_specs=[
        pl.BlockSpec((RB, F), lambda i: (i, 0)),
        pl.BlockSpec((NC, RB, F), lambda i: (0, i, 0)),
        pl.BlockSpec((RB, F), lambda i: (i, 0)),
        pl.BlockSpec((1, F), lambda i: (0, 0)),
        pl.BlockSpec((F, F), lambda i: (0, 0)),
    ],
    out_specs=pl.BlockSpec((RB, F), lambda i: (i, 0)),
    out_shape=jax.ShapeDtypeStruct((N_PAD, F), jnp.float32),
)


def _tc_fin_body(t_ref, p_ref, dinv_ref, b_ref, wp0_ref, bp0_ref,
                 wp1_ref, bp1_ref, emb_ref, logp_ref):
    s = t_ref[...] + p_ref[0] + p_ref[1]
    emb = dinv_ref[...] * s + b_ref[...]
    emb_ref[...] = emb
    h = jnp.tanh(emb)
    y = jnp.dot(h, wp0_ref[...], preferred_element_type=jnp.float32) + bp0_ref[...]
    y = jnp.dot(y, wp1_ref[...], preferred_element_type=jnp.float32) + bp1_ref[...]
    m = jnp.max(y, axis=1, keepdims=True)
    e = y - m
    logp_ref[...] = e - jnp.log(jnp.sum(jnp.exp(e), axis=1, keepdims=True))


_tc_fin = pl.pallas_call(
    _tc_fin_body,
    grid=(GRID,),
    in_specs=[
        pl.BlockSpec((RB, F), lambda i: (i, 0)),
        pl.BlockSpec((NC, RB, F), lambda i: (0, i, 0)),
        pl.BlockSpec((RB, F), lambda i: (i, 0)),
        pl.BlockSpec((1, F), lambda i: (0, 0)),
        pl.BlockSpec((F, F), lambda i: (0, 0)),
        pl.BlockSpec((1, F), lambda i: (0, 0)),
        pl.BlockSpec((F, OUT), lambda i: (0, 0)),
        pl.BlockSpec((1, OUT), lambda i: (0, 0)),
    ],
    out_specs=[
        pl.BlockSpec((RB, F), lambda i: (i, 0)),
        pl.BlockSpec((RB, OUT), lambda i: (i, 0)),
    ],
    out_shape=[
        jax.ShapeDtypeStruct((N_PAD, F), jnp.float32),
        jax.ShapeDtypeStruct((N_PAD, OUT), jnp.float32),
    ],
)


def kernel(x, edge_index, batch, W0, b0, W1, b1, W2, b2, Wp0, bp0, Wp1, bp1):
    e_total = edge_index.shape[1]
    n_chunks = -(-e_total // (NW * CHUNK))
    m = PHASES * KG
    n_chunks = -(-n_chunks // m) * m
    e_pad = NW * n_chunks * CHUNK

    src = edge_index[0]
    dst = edge_index[1]
    pad = e_pad - e_total
    pad_idx = jnp.full((pad,), N, dtype=jnp.int32)
    cpp = n_chunks // PHASES
    src_r = jnp.concatenate([src, pad_idx]).reshape(NW, PHASES, cpp * CHUNK)
    dst_r = jnp.concatenate([dst, pad_idx]).reshape(NW, PHASES, cpp, CHUNK)

    x_p = jnp.concatenate(
        [x, jnp.zeros((N_PAD - N, F), dtype=jnp.float32)], axis=0)
    zeros_f = jnp.zeros((N_PAD, F), dtype=jnp.float32)
    ones_c = jnp.ones((CHUNK, DEG_W), dtype=jnp.float32)

    deg_fn = _make_deg(n_chunks)
    prop_fn = _make_prop(n_chunks)

    degp = deg_fn(dst_r, zeros_f, ones_c)
    t0, dinvb = _tc0(x_p, W0, degp)
    p1 = prop_fn(t0, src_r, dst_r, zeros_f)
    t1 = _tc_mid(t0, p1, dinvb, b0.reshape(1, F), W1)
    p2 = prop_fn(t1, src_r, dst_r, zeros_f)
    t2 = _tc_mid(t1, p2, dinvb, b1.reshape(1, F), W2)
    p3 = prop_fn(t2, src_r, dst_r, zeros_f)
    emb_p, logp_p = _tc_fin(t2, p3, dinvb, b2.reshape(1, F),
                            Wp0, bp0.reshape(1, F), Wp1, bp1.reshape(1, OUT))
    return emb_p[:N], logp_p[:N]


# speedup vs baseline: 1.8928x; 1.8928x over previous
import functools

import jax
import jax.numpy as jnp
import numpy as np
from jax import lax
from jax.experimental import pallas as pl
from jax.experimental.pallas import tpu as pltpu
from jax.experimental.pallas import tpu_sc as plsc

N = 10000
F = 128
OUT = 64
N_PAD = 10240
NC, NS = 2, 16
NW = NC * NS
ROWS_PER_TILE = N_PAD // NS
CHUNK = 128
DEG_W = F
CORE0_FRAC = 0.65

_MESH = plsc.VectorSubcoreMesh(core_axis_name="c", subcore_axis_name="s")


def _make_deg(ca, cb, cmax):
    @functools.partial(
        pl.kernel,
        out_type=jax.ShapeDtypeStruct((NC, N_PAD, DEG_W), jnp.float32),
        mesh=_MESH,
        scratch_types=[
            pltpu.VMEM_SHARED((N_PAD, DEG_W), jnp.float32),
            pltpu.VMEM((cmax, CHUNK), jnp.int32),
            pltpu.VMEM((CHUNK, DEG_W), jnp.float32),
        ],
    )
    def deg_kernel(dst_hbm, zeros_hbm, ones_hbm, out_hbm, acc, dst_v, ones_v):
        cid = lax.axis_index("c")
        sid = lax.axis_index("s")
        wid = sid * NC + cid
        row0 = sid * ROWS_PER_TILE
        pltpu.sync_copy(zeros_hbm.at[pl.ds(row0, ROWS_PER_TILE)],
                        acc.at[pl.ds(row0, ROWS_PER_TILE)])
        pltpu.sync_copy(dst_hbm.at[wid], dst_v)
        pltpu.sync_copy(ones_hbm, ones_v)
        plsc.subcore_barrier()

        def body(j, carry):
            pltpu.sync_copy(ones_v, acc.at[dst_v.at[j]], add=True)
            return carry

        @pl.when(cid == 0)
        def _():
            lax.fori_loop(0, ca, body, 0)

        @pl.when(cid == 1)
        def _():
            lax.fori_loop(0, cb, body, 0)

        plsc.subcore_barrier()
        pltpu.sync_copy(acc.at[pl.ds(row0, ROWS_PER_TILE)],
                        out_hbm.at[cid, pl.ds(row0, ROWS_PER_TILE)])

    return deg_kernel


def _make_prop(ca, cb, cmax):
    @functools.partial(
        pl.kernel,
        out_type=jax.ShapeDtypeStruct((NC, N_PAD, F), jnp.float32),
        mesh=_MESH,
        scratch_types=[
            pltpu.VMEM_SHARED((N_PAD, F), jnp.float32),
            pltpu.VMEM((cmax, CHUNK), jnp.int32),
            pltpu.VMEM((cmax, CHUNK), jnp.int32),
            pltpu.VMEM((CHUNK, F), jnp.float32),
            pltpu.SemaphoreType.DMA,
        ],
    )
    def prop_kernel(t_hbm, src_hbm, dst_hbm, zeros_hbm, out_hbm,
                    acc, src_v, dst_v, rows_v, sem):
        cid = lax.axis_index("c")
        sid = lax.axis_index("s")
        wid = sid * NC + cid
        row0 = sid * ROWS_PER_TILE
        pltpu.sync_copy(zeros_hbm.at[pl.ds(row0, ROWS_PER_TILE)],
                        acc.at[pl.ds(row0, ROWS_PER_TILE)])
        pltpu.sync_copy(src_hbm.at[wid], src_v)
        pltpu.sync_copy(dst_hbm.at[wid], dst_v)
        plsc.subcore_barrier()

        def body(j, carry):
            pltpu.async_copy(t_hbm.at[src_v.at[j]], rows_v, sem).wait()
            pltpu.sync_copy(rows_v, acc.at[dst_v.at[j]], add=True)
            return carry

        @pl.when(cid == 0)
        def _():
            lax.fori_loop(0, ca, body, 0)

        @pl.when(cid == 1)
        def _():
            lax.fori_loop(0, cb, body, 0)

        plsc.subcore_barrier()
        pltpu.sync_copy(acc.at[pl.ds(row0, ROWS_PER_TILE)],
                        out_hbm.at[cid, pl.ds(row0, ROWS_PER_TILE)])

    return prop_kernel


RB = 1280
GRID = N_PAD // RB


def _tc0_body(x_ref, w_ref, p_ref, t_ref, dinv_ref):
    p = p_ref[...]
    deg = 1.0 + p[0, :, 0:1] + p[1, :, 0:1]
    dinvb = jnp.broadcast_to(lax.rsqrt(deg), (RB, F))
    t_ref[...] = dinvb * jnp.dot(x_ref[...], w_ref[...],
                                 preferred_element_type=jnp.float32)
    dinv_ref[...] = dinvb


_tc0 = pl.pallas_call(
    _tc0_body,
    grid=(GRID,),
    in_specs=[
        pl.BlockSpec((RB, F), lambda i: (i, 0)),
        pl.BlockSpec((F, F), lambda i: (0, 0)),
        pl.BlockSpec((NC, RB, DEG_W), lambda i: (0, i, 0)),
    ],
    out_specs=[
        pl.BlockSpec((RB, F), lambda i: (i, 0)),
        pl.BlockSpec((RB, F), lambda i: (i, 0)),
    ],
    out_shape=[
        jax.ShapeDtypeStruct((N_PAD, F), jnp.float32),
        jax.ShapeDtypeStruct((N_PAD, F), jnp.float32),
    ],
)


def _tc_mid_body(t_ref, p_ref, dinv_ref, b_ref, w_ref, out_ref):
    s = t_ref[...] + p_ref[0] + p_ref[1]
    pre = dinv_ref[...] * s + b_ref[...]
    h = jnp.tanh(pre)
    out_ref[...] = dinv_ref[...] * jnp.dot(h, w_ref[...],
                                           preferred_element_type=jnp.float32)


_tc_mid = pl.pallas_call(
    _tc_mid_body,
    grid=(GRID,),
    in_specs=[
        pl.BlockSpec((RB, F), lambda i: (i, 0)),
        pl.BlockSpec((NC, RB, F), lambda i: (0, i, 0)),
        pl.BlockSpec((RB, F), lambda i: (i, 0)),
        pl.BlockSpec((1, F), lambda i: (0, 0)),
        pl.BlockSpec((F, F), lambda i: (0, 0)),
    ],
    out_specs=pl.BlockSpec((RB, F), lambda i: (i, 0)),
    out_shape=jax.ShapeDtypeStruct((N_PAD, F), jnp.float32),
)


def _tc_fin_body(t_ref, p_ref, dinv_ref, b_ref, wp0_ref, bp0_ref,
                 wp1_ref, bp1_ref, emb_ref, logp_ref):
    s = t_ref[...] + p_ref[0] + p_ref[1]
    emb = dinv_ref[...] * s + b_ref[...]
    emb_ref[...] = emb
    h = jnp.tanh(emb)
    y = jnp.dot(h, wp0_ref[...], preferred_element_type=jnp.float32) + bp0_ref[...]
    y = jnp.dot(y, wp1_ref[...], preferred_element_type=jnp.float32) + bp1_ref[...]
    m = jnp.max(y, axis=1, keepdims=True)
    e = y - m
    logp_ref[...] = e - jnp.log(jnp.sum(jnp.exp(e), axis=1, keepdims=True))


_tc_fin = pl.pallas_call(
    _tc_fin_body,
    grid=(GRID,),
    in_specs=[
        pl.BlockSpec((RB, F), lambda i: (i, 0)),
        pl.BlockSpec((NC, RB, F), lambda i: (0, i, 0)),
        pl.BlockSpec((RB, F), lambda i: (i, 0)),
        pl.BlockSpec((1, F), lambda i: (0, 0)),
        pl.BlockSpec((F, F), lambda i: (0, 0)),
        pl.BlockSpec((1, F), lambda i: (0, 0)),
        pl.BlockSpec((F, OUT), lambda i: (0, 0)),
        pl.BlockSpec((1, OUT), lambda i: (0, 0)),
    ],
    out_specs=[
        pl.BlockSpec((RB, F), lambda i: (i, 0)),
        pl.BlockSpec((RB, OUT), lambda i: (i, 0)),
    ],
    out_shape=[
        jax.ShapeDtypeStruct((N_PAD, F), jnp.float32),
        jax.ShapeDtypeStruct((N_PAD, OUT), jnp.float32),
    ],
)


def _chunk_layout(e_total):
    n_real = -(-e_total // CHUNK)
    per_pair = -(-n_real // NS)
    ca = max(1, int(round(per_pair * CORE0_FRAC)))
    cb = per_pair - ca
    cmax = max(ca, cb)
    counts = [ca if w % 2 == 0 else cb for w in range(NW)]
    seg_end = np.cumsum(counts)
    seg_start = seg_end - np.asarray(counts)
    n_seg = int(seg_end[-1])
    n_rows = n_seg + 1
    perm = []
    for w in range(NW):
        rows = list(range(int(seg_start[w]), int(seg_end[w])))
        rows += [n_seg] * (cmax - counts[w])
        perm.append(rows)
    return ca, cb, cmax, np.asarray(perm, dtype=np.int32), n_rows


def kernel(x, edge_index, batch, W0, b0, W1, b1, W2, b2, Wp0, bp0, Wp1, bp1):
    e_total = edge_index.shape[1]
    ca, cb, cmax, perm, n_rows = _chunk_layout(e_total)
    e_pad = n_rows * CHUNK

    src = edge_index[0]
    dst = edge_index[1]
    pad_idx = jnp.full((e_pad - e_total,), N, dtype=jnp.int32)
    src_r = jnp.take(jnp.concatenate([src, pad_idx]).reshape(n_rows, CHUNK),
                     perm.reshape(-1), axis=0).reshape(NW, cmax, CHUNK)
    dst_r = jnp.take(jnp.concatenate([dst, pad_idx]).reshape(n_rows, CHUNK),
                     perm.reshape(-1), axis=0).reshape(NW, cmax, CHUNK)

    x_p = jnp.concatenate(
        [x, jnp.zeros((N_PAD - N, F), dtype=jnp.float32)], axis=0)
    zeros_f = jnp.zeros((N_PAD, F), dtype=jnp.float32)
    ones_c = jnp.ones((CHUNK, DEG_W), dtype=jnp.float32)

    deg_fn = _make_deg(ca, cb, cmax)
    prop_fn = _make_prop(ca, cb, cmax)

    degp = deg_fn(dst_r, zeros_f, ones_c)
    t0, dinvb = _tc0(x_p, W0, degp)
    p1 = prop_fn(t0, src_r, dst_r, zeros_f)
    t1 = _tc_mid(t0, p1, dinvb, b0.reshape(1, F), W1)
    p2 = prop_fn(t1, src_r, dst_r, zeros_f)
    t2 = _tc_mid(t1, p2, dinvb, b1.reshape(1, F), W2)
    p3 = prop_fn(t2, src_r, dst_r, zeros_f)
    emb_p, logp_p = _tc_fin(t2, p3, dinvb, b2.reshape(1, F),
                            Wp0, bp0.reshape(1, F), Wp1, bp1.reshape(1, OUT))
    return emb_p[:N], logp_p[:N]


# speedup vs baseline: 1.9081x; 1.0081x over previous
import functools

import jax
import jax.numpy as jnp
import numpy as np
from jax import lax
from jax.experimental import pallas as pl
from jax.experimental.pallas import tpu as pltpu
from jax.experimental.pallas import tpu_sc as plsc

N = 10000
F = 128
OUT = 64
N_PAD = 10240
NC, NS = 2, 16
NW = NC * NS
ROWS_PER_TILE = N_PAD // NS
CHUNK = 128
DEG_W = F
CORE0_FRAC = 0.70

_MESH = plsc.VectorSubcoreMesh(core_axis_name="c", subcore_axis_name="s")


def _make_deg(ca, cb, cmax):
    @functools.partial(
        pl.kernel,
        out_type=jax.ShapeDtypeStruct((NC, N_PAD, DEG_W), jnp.float32),
        mesh=_MESH,
        scratch_types=[
            pltpu.VMEM_SHARED((N_PAD, DEG_W), jnp.float32),
            pltpu.VMEM((cmax, CHUNK), jnp.int32),
            pltpu.VMEM((CHUNK, DEG_W), jnp.float32),
        ],
    )
    def deg_kernel(dsta_hbm, dstb_hbm, zeros_hbm, ones_hbm, out_hbm,
                   acc, dst_v, ones_v):
        cid = lax.axis_index("c")
        sid = lax.axis_index("s")
        row0 = sid * ROWS_PER_TILE
        pltpu.sync_copy(zeros_hbm.at[pl.ds(row0, ROWS_PER_TILE)],
                        acc.at[pl.ds(row0, ROWS_PER_TILE)])

        @pl.when(cid == 0)
        def _():
            pltpu.sync_copy(dsta_hbm.at[sid], dst_v.at[pl.ds(0, ca)])

        @pl.when(cid == 1)
        def _():
            pltpu.sync_copy(dstb_hbm.at[sid], dst_v.at[pl.ds(0, cb)])

        pltpu.sync_copy(ones_hbm, ones_v)
        plsc.subcore_barrier()

        def body(j, carry):
            pltpu.sync_copy(ones_v, acc.at[dst_v.at[j]], add=True)
            return carry

        @pl.when(cid == 0)
        def _():
            lax.fori_loop(0, ca, body, 0)

        @pl.when(cid == 1)
        def _():
            lax.fori_loop(0, cb, body, 0)

        plsc.subcore_barrier()
        pltpu.sync_copy(acc.at[pl.ds(row0, ROWS_PER_TILE)],
                        out_hbm.at[cid, pl.ds(row0, ROWS_PER_TILE)])

    return deg_kernel


def _make_prop(ca, cb, cmax):
    @functools.partial(
        pl.kernel,
        out_type=jax.ShapeDtypeStruct((NC, N_PAD, F), jnp.float32),
        mesh=_MESH,
        scratch_types=[
            pltpu.VMEM_SHARED((N_PAD, F), jnp.float32),
            pltpu.VMEM((cmax, CHUNK), jnp.int32),
            pltpu.VMEM((cmax, CHUNK), jnp.int32),
            pltpu.VMEM((CHUNK, F), jnp.float32),
            pltpu.SemaphoreType.DMA,
        ],
    )
    def prop_kernel(t_hbm, srca_hbm, srcb_hbm, dsta_hbm, dstb_hbm,
                    zeros_hbm, out_hbm, acc, src_v, dst_v, rows_v, sem):
        cid = lax.axis_index("c")
        sid = lax.axis_index("s")
        row0 = sid * ROWS_PER_TILE
        pltpu.sync_copy(zeros_hbm.at[pl.ds(row0, ROWS_PER_TILE)],
                        acc.at[pl.ds(row0, ROWS_PER_TILE)])

        @pl.when(cid == 0)
        def _():
            pltpu.sync_copy(srca_hbm.at[sid], src_v.at[pl.ds(0, ca)])
            pltpu.sync_copy(dsta_hbm.at[sid], dst_v.at[pl.ds(0, ca)])

        @pl.when(cid == 1)
        def _():
            pltpu.sync_copy(srcb_hbm.at[sid], src_v.at[pl.ds(0, cb)])
            pltpu.sync_copy(dstb_hbm.at[sid], dst_v.at[pl.ds(0, cb)])

        plsc.subcore_barrier()

        def body(j, carry):
            pltpu.async_copy(t_hbm.at[src_v.at[j]], rows_v, sem).wait()
            pltpu.sync_copy(rows_v, acc.at[dst_v.at[j]], add=True)
            return carry

        @pl.when(cid == 0)
        def _():
            lax.fori_loop(0, ca, body, 0)

        @pl.when(cid == 1)
        def _():
            lax.fori_loop(0, cb, body, 0)

        plsc.subcore_barrier()
        pltpu.sync_copy(acc.at[pl.ds(row0, ROWS_PER_TILE)],
                        out_hbm.at[cid, pl.ds(row0, ROWS_PER_TILE)])

    return prop_kernel


RB = 1280
GRID = N_PAD // RB


def _tc0_body(x_ref, w_ref, p_ref, t_ref, dinv_ref):
    p = p_ref[...]
    deg = 1.0 + p[0, :, 0:1] + p[1, :, 0:1]
    dinvb = jnp.broadcast_to(lax.rsqrt(deg), (RB, F))
    t_ref[...] = dinvb * jnp.dot(x_ref[...], w_ref[...],
                                 preferred_element_type=jnp.float32)
    dinv_ref[...] = dinvb


_tc0 = pl.pallas_call(
    _tc0_body,
    grid=(GRID,),
    in_specs=[
        pl.BlockSpec((RB, F), lambda i: (i, 0)),
        pl.BlockSpec((F, F), lambda i: (0, 0)),
        pl.BlockSpec((NC, RB, DEG_W), lambda i: (0, i, 0)),
    ],
    out_specs=[
        pl.BlockSpec((RB, F), lambda i: (i, 0)),
        pl.BlockSpec((RB, F), lambda i: (i, 0)),
    ],
    out_shape=[
        jax.ShapeDtypeStruct((N_PAD, F), jnp.float32),
        jax.ShapeDtypeStruct((N_PAD, F), jnp.float32),
    ],
)


def _tc_mid_body(t_ref, p_ref, dinv_ref, b_ref, w_ref, out_ref):
    s = t_ref[...] + p_ref[0] + p_ref[1]
    pre = dinv_ref[...] * s + b_ref[...]
    h = jnp.tanh(pre)
    out_ref[...] = dinv_ref[...] * jnp.dot(h, w_ref[...],
                                           preferred_element_type=jnp.float32)


_tc_mid = pl.pallas_call(
    _tc_mid_body,
    grid=(GRID,),
    in_specs=[
        pl.BlockSpec((RB, F), lambda i: (i, 0)),
        pl.BlockSpec((NC, RB, F), lambda i: (0, i, 0)),
        pl.BlockSpec((RB, F), lambda i: (i, 0)),
        pl.BlockSpec((1, F), lambda i: (0, 0)),
        pl.BlockSpec((F, F), lambda i: (0, 0)),
    ],
    out_specs=pl.BlockSpec((RB, F), lambda i: (i, 0)),
    out_shape=jax.ShapeDtypeStruct((N_PAD, F), jnp.float32),
)


def _tc_fin_body(t_ref, p_ref, dinv_ref, b_ref, wp0_ref, bp0_ref,
                 wp1_ref, bp1_ref, emb_ref, logp_ref):
    s = t_ref[...] + p_ref[0] + p_ref[1]
    emb = dinv_ref[...] * s + b_ref[...]
    emb_ref[...] = emb
    h = jnp.tanh(emb)
    y = jnp.dot(h, wp0_ref[...], preferred_element_type=jnp.float32) + bp0_ref[...]
    y = jnp.dot(y, wp1_ref[...], preferred_element_type=jnp.float32) + bp1_ref[...]
    m = jnp.max(y, axis=1, keepdims=True)
    e = y - m
    logp_ref[...] = e - jnp.log(jnp.sum(jnp.exp(e), axis=1, keepdims=True))


_tc_fin = pl.pallas_call(
    _tc_fin_body,
    grid=(GRID,),
    in_specs=[
        pl.BlockSpec((RB, F), lambda i: (i, 0)),
        pl.BlockSpec((NC, RB, F), lambda i: (0, i, 0)),
        pl.BlockSpec((RB, F), lambda i: (i, 0)),
        pl.BlockSpec((1, F), lambda i: (0, 0)),
        pl.BlockSpec((F, F), lambda i: (0, 0)),
        pl.BlockSpec((1, F), lambda i: (0, 0)),
        pl.BlockSpec((F, OUT), lambda i: (0, 0)),
        pl.BlockSpec((1, OUT), lambda i: (0, 0)),
    ],
    out_specs=[
        pl.BlockSpec((RB, F), lambda i: (i, 0)),
        pl.BlockSpec((RB, OUT), lambda i: (i, 0)),
    ],
    out_shape=[
        jax.ShapeDtypeStruct((N_PAD, F), jnp.float32),
        jax.ShapeDtypeStruct((N_PAD, OUT), jnp.float32),
    ],
)


def _chunk_layout(e_total):
    n_real = -(-e_total // CHUNK)
    per_pair = -(-n_real // NS)
    ca = max(1, int(round(per_pair * CORE0_FRAC)))
    cb = per_pair - ca
    return ca, cb, max(ca, cb), NS * (ca + cb)


def kernel(x, edge_index, batch, W0, b0, W1, b1, W2, b2, Wp0, bp0, Wp1, bp1):
    e_total = edge_index.shape[1]
    ca, cb, cmax, n_rows = _chunk_layout(e_total)
    e_pad = n_rows * CHUNK

    src = edge_index[0]
    dst = edge_index[1]
    pad_idx = jnp.full((e_pad - e_total,), N, dtype=jnp.int32)
    src_f = jnp.concatenate([src, pad_idx]).reshape(n_rows, CHUNK)
    dst_f = jnp.concatenate([dst, pad_idx]).reshape(n_rows, CHUNK)
    na = NS * ca
    src_a = src_f[:na].reshape(NS, ca, CHUNK)
    src_b = src_f[na:].reshape(NS, cb, CHUNK)
    dst_a = dst_f[:na].reshape(NS, ca, CHUNK)
    dst_b = dst_f[na:].reshape(NS, cb, CHUNK)

    x_p = jnp.concatenate(
        [x, jnp.zeros((N_PAD - N, F), dtype=jnp.float32)], axis=0)
    zeros_f = jnp.zeros((N_PAD, F), dtype=jnp.float32)
    ones_c = jnp.ones((CHUNK, DEG_W), dtype=jnp.float32)

    deg_fn = _make_deg(ca, cb, cmax)
    prop_fn = _make_prop(ca, cb, cmax)

    degp = deg_fn(dst_a, dst_b, zeros_f, ones_c)
    t0, dinvb = _tc0(x_p, W0, degp)
    p1 = prop_fn(t0, src_a, src_b, dst_a, dst_b, zeros_f)
    t1 = _tc_mid(t0, p1, dinvb, b0.reshape(1, F), W1)
    p2 = prop_fn(t1, src_a, src_b, dst_a, dst_b, zeros_f)
    t2 = _tc_mid(t1, p2, dinvb, b1.reshape(1, F), W2)
    p3 = prop_fn(t2, src_a, src_b, dst_a, dst_b, zeros_f)
    emb_p, logp_p = _tc_fin(t2, p3, dinvb, b2.reshape(1, F),
                            Wp0, bp0.reshape(1, F), Wp1, bp1.reshape(1, OUT))
    return emb_p[:N], logp_p[:N]
